# baseline (speedup 1.0000x reference)
import functools

import jax
import jax.numpy as jnp
from jax import lax
from jax.experimental import pallas as pl
from jax.experimental.pallas import tpu as pltpu
from jax.experimental.pallas import tpu_sc as plsc

F32 = jnp.float32

N = 10000
E = 320000
D = 128
G = 50
NMOL = 100
CUTOFF = 5.0
WIDTH = CUTOFF / (G - 1)
LN2 = 0.6931471805599453

NC = 2
NS = 16
NW = NC * NS
EPW = 10240
E_PAD = NW * EPW
K = 128
KD = EPW // K
KC = 40
NCH = EPW // KC
N_PAD = 10112
RPT = N_PAD // NS

BN = 1000
BE_W = 1024


def _ssp(x):
    return jnp.maximum(x, 0.0) + jnp.log(1.0 + jnp.exp(-jnp.abs(x))) - LN2


def _embed_body(z_ref, emb_ref, wn_ref, bn_ref, r_ref, rn_ref):
    z = z_ref[...]
    ids = lax.broadcasted_iota(jnp.int32, (BN, 100), 1).astype(F32)
    oh = (z == ids).astype(F32)
    r = jnp.dot(oh, emb_ref[...], preferred_element_type=F32)
    r_ref[...] = r
    rn_ref[...] = jnp.dot(r, wn_ref[...], preferred_element_type=F32) + bn_ref[...]


def _wfilt_body(d2_ref, w1_ref, b1_ref, w2_ref, b2_ref, out_ref):
    pid = pl.program_id(0)
    d = jnp.sqrt(d2_ref[...])
    offs = lax.broadcasted_iota(jnp.int32, (BE_W, G), 1).astype(F32) * WIDTH
    g = jnp.exp(-0.5 * ((d - offs) / WIDTH) ** 2)
    h = _ssp(jnp.dot(g, w1_ref[...], preferred_element_type=F32) + b1_ref[...])
    w = jnp.dot(h, w2_ref[...], preferred_element_type=F32) + b2_ref[...]
    eid = pid * BE_W + lax.broadcasted_iota(jnp.int32, (BE_W, 1), 0)
    out_ref[...] = jnp.where(eid < E, w, 0.0)


def _update_body(a0_ref, a1_ref, r_ref, wu1_ref, bu1_ref, wu2_ref, bu2_ref,
                 wn_ref, bn_ref, r2_ref, rn_ref):
    agg = a0_ref[...] + a1_ref[...]
    t = _ssp(jnp.dot(agg, wu1_ref[...], preferred_element_type=F32) + bu1_ref[...])
    r2 = r_ref[...] + jnp.dot(t, wu2_ref[...], preferred_element_type=F32) + bu2_ref[...]
    r2_ref[...] = r2
    rn_ref[...] = jnp.dot(r2, wn_ref[...], preferred_element_type=F32) + bn_ref[...]


def _update_readout_body(a0_ref, a1_ref, r_ref, wu1_ref, bu1_ref, wu2_ref,
                         bu2_ref, wr1_ref, br1_ref, wr2_ref, br2_ref, aw_ref):
    agg = a0_ref[...] + a1_ref[...]
    t = _ssp(jnp.dot(agg, wu1_ref[...], preferred_element_type=F32) + bu1_ref[...])
    r2 = r_ref[...] + jnp.dot(t, wu2_ref[...], preferred_element_type=F32) + bu2_ref[...]
    t2 = _ssp(jnp.dot(r2, wr1_ref[...], preferred_element_type=F32) + br1_ref[...])
    aw_ref[...] = jnp.dot(t2, wr2_ref[...], preferred_element_type=F32) + br2_ref[...]


def _pool_body(a_ref, e_ref):
    e_ref[...] = jnp.sum(a_ref[...], axis=1, keepdims=True)


def _full(shape):
    return pl.BlockSpec(shape, lambda i: tuple(0 for _ in shape))


def _embed_call(z_col, emb, wn, bn):
    return pl.pallas_call(
        _embed_body,
        grid=(N // BN,),
        in_specs=[pl.BlockSpec((BN, 1), lambda i: (i, 0)),
                  _full((100, D)), _full((D, D)), _full((1, D))],
        out_specs=[pl.BlockSpec((BN, D), lambda i: (i, 0)),
                   pl.BlockSpec((BN, D), lambda i: (i, 0))],
        out_shape=[jax.ShapeDtypeStruct((N, D), F32),
                   jax.ShapeDtypeStruct((N, D), F32)],
    )(z_col, emb, wn, bn)


def _wfilt_call(dist, w1, b1, w2, b2):
    return pl.pallas_call(
        _wfilt_body,
        grid=(E_PAD // BE_W,),
        in_specs=[pl.BlockSpec((BE_W, 1), lambda i: (i, 0)),
                  _full((G, D)), _full((1, D)), _full((D, D)), _full((1, D))],
        out_specs=pl.BlockSpec((BE_W, D), lambda i: (i, 0)),
        out_shape=jax.ShapeDtypeStruct((E_PAD, D), F32),
    )(dist, w1, b1, w2, b2)


def _update_call(a0, a1, r, wu1, bu1, wu2, bu2, wn, bn):
    return pl.pallas_call(
        _update_body,
        grid=(N // BN,),
        in_specs=[pl.BlockSpec((BN, D), lambda i: (i, 0)),
                  pl.BlockSpec((BN, D), lambda i: (i, 0)),
                  pl.BlockSpec((BN, D), lambda i: (i, 0)),
                  _full((D, D)), _full((1, D)), _full((D, D)), _full((1, D)),
                  _full((D, D)), _full((1, D))],
        out_specs=[pl.BlockSpec((BN, D), lambda i: (i, 0)),
                   pl.BlockSpec((BN, D), lambda i: (i, 0))],
        out_shape=[jax.ShapeDtypeStruct((N, D), F32),
                   jax.ShapeDtypeStruct((N, D), F32)],
    )(a0, a1, r, wu1, bu1, wu2, bu2, wn, bn)


def _update_readout_call(a0, a1, r, wu1, bu1, wu2, bu2, wr1, br1, wr2, br2):
    return pl.pallas_call(
        _update_readout_body,
        grid=(N // BN,),
        in_specs=[pl.BlockSpec((BN, D), lambda i: (i, 0)),
                  pl.BlockSpec((BN, D), lambda i: (i, 0)),
                  pl.BlockSpec((BN, D), lambda i: (i, 0)),
                  _full((D, D)), _full((1, D)), _full((D, D)), _full((1, D)),
                  _full((D, D // 2)), _full((1, D // 2)),
                  _full((D // 2, 1)), _full((1, 1))],
        out_specs=pl.BlockSpec((BN, 1), lambda i: (i, 0)),
        out_shape=jax.ShapeDtypeStruct((N, 1), F32),
    )(a0, a1, r, wu1, bu1, wu2, bu2, wr1, br1, wr2, br2)


def _pool_call(aw):
    return pl.pallas_call(
        _pool_body,
        grid=(1,),
        in_specs=[_full((NMOL, NMOL))],
        out_specs=_full((NMOL, 1)),
        out_shape=jax.ShapeDtypeStruct((NMOL, 1), F32),
    )(aw)


_SC_MESH = plsc.VectorSubcoreMesh(core_axis_name="c", subcore_axis_name="s",
                                  num_cores=NC, num_subcores=NS)


@functools.partial(
    pl.kernel,
    out_type=jax.ShapeDtypeStruct((E_PAD,), F32),
    mesh=_SC_MESH,
    scratch_types=[
        pltpu.VMEM((N * 4,), F32),
        pltpu.VMEM((K,), jnp.int32),
        pltpu.VMEM((K,), jnp.int32),
        pltpu.VMEM((K,), F32),
    ],
    compiler_params=pltpu.CompilerParams(needs_layout_passes=False),
)
def _sc_dist2(xyz_hbm, i_hbm, j_hbm, d2_hbm, xyz_v, ii_v, jj_v, d_v):
    cc = lax.axis_index("c")
    ss = lax.axis_index("s")
    base = (ss * NC + cc) * EPW
    pltpu.sync_copy(xyz_hbm, xyz_v)

    def chunk(ci, carry):
        off = base + ci * K
        pltpu.sync_copy(i_hbm.at[pl.ds(off, K)], ii_v)
        pltpu.sync_copy(j_hbm.at[pl.ds(off, K)], jj_v)
        for k in range(8):
            sl = pl.ds(16 * k, 16)
            vi = ii_v[sl] * 4
            vj = jj_v[sl] * 4
            acc = jnp.zeros((16,), F32)
            for ax in range(3):
                a = plsc.load_gather(xyz_v, [vi + ax])
                b = plsc.load_gather(xyz_v, [vj + ax])
                dd = a - b
                acc = acc + dd * dd
            d_v[sl] = acc
        pltpu.sync_copy(d_v, d2_hbm.at[pl.ds(off, K)])
        return carry

    lax.fori_loop(0, KD, chunk, 0)


@functools.partial(
    pl.kernel,
    out_type=jax.ShapeDtypeStruct((NC, N, D), F32),
    mesh=_SC_MESH,
    scratch_types=[
        pltpu.VMEM((KC, D), F32),
        pltpu.VMEM((KC, D), F32),
        pltpu.VMEM((KC, D), F32),
        pltpu.VMEM((KC, D), F32),
        pltpu.VMEM((KC, D), F32),
        pltpu.VMEM((KC, D), F32),
        pltpu.VMEM((8, KC), jnp.int32),
        pltpu.VMEM((8, KC), jnp.int32),
        pltpu.VMEM((8, KC), jnp.int32),
        pltpu.VMEM((8, KC), jnp.int32),
        pltpu.VMEM_SHARED((N, D), F32),
        pltpu.SemaphoreType.DMA,
        pltpu.SemaphoreType.DMA,
        pltpu.SemaphoreType.DMA,
        pltpu.SemaphoreType.DMA,
        pltpu.SemaphoreType.DMA,
        pltpu.SemaphoreType.DMA,
    ],
)
def _sc_conv(i2_hbm, j2_hbm, w_hbm, rn_hbm, out_hbm,
             rj0, ri0, w0, rj1, ri1, w1, iiA, jjA, iiB, jjB,
             agg_sh, semg0, semg1, sems0, sems1, semiA, semiB):
    sets = ((rj0, ri0, w0, semg0, sems0),
            (rj1, ri1, w1, semg1, sems1))
    cc = lax.axis_index("c")
    ss = lax.axis_index("s")
    base = (ss * NC + cc) * EPW
    baser = (ss * NC + cc) * NCH
    NG = N // 8

    def zrow(t, carry):
        for k in range(8):
            rj0[t, pl.ds(16 * k, 16)] = jnp.zeros((16,), F32)
        return carry

    lax.fori_loop(0, 8, zrow, 0)

    def zcopy(q, carry):
        g = ss + 16 * q

        @pl.when(g < NG)
        def _():
            pltpu.sync_copy(rj0.at[pl.ds(0, 8)], agg_sh.at[pl.ds(g * 8, 8)])

        return carry

    lax.fori_loop(0, (NG + 15) // 16, zcopy, 0)
    plsc.subcore_barrier()

    def issue_in(S, ci, ir, jr):
        rj, ri, w, semg, _ = sets[S]
        off = base + ci * KC
        pltpu.async_copy(rn_hbm.at[jr], rj, semg)
        pltpu.async_copy(rn_hbm.at[ir], ri, semg)
        pltpu.async_copy(w_hbm.at[pl.ds(off, KC)], w, semg)

    def wait_in(S, ci, ir, jr):
        rj, ri, w, semg, _ = sets[S]
        off = base + ci * KC
        pltpu.make_async_copy(rn_hbm.at[jr], rj, semg).wait()
        pltpu.make_async_copy(rn_hbm.at[ir], ri, semg).wait()
        pltpu.make_async_copy(w_hbm.at[pl.ds(off, KC)], w, semg).wait()

    def compute(S):
        rj, ri, w = sets[S][0], sets[S][1], sets[S][2]

        def edges(t, carry):
            for u in range(4):
                e = t * 4 + u
                for k in range(8):
                    sl = pl.ds(16 * k, 16)
                    wv = w[e, sl]
                    rj[e, sl] = rj[e, sl] * wv
                    ri[e, sl] = ri[e, sl] * wv
            return carry

        lax.fori_loop(0, KC // 4, edges, 0)

    def issue_out(S, ir, jr):
        rj, ri, _, _, sems = sets[S]
        pltpu.async_copy(rj, agg_sh.at[ir], sems, add=True)
        pltpu.async_copy(ri, agg_sh.at[jr], sems, add=True)

    def wait_out(S, ir, jr):
        rj, ri, _, _, sems = sets[S]
        pltpu.make_async_copy(rj, agg_sh.at[ir], sems).wait()
        pltpu.make_async_copy(ri, agg_sh.at[jr], sems).wait()

    def idxrow(s):
        if s < 8:
            return iiA.at[s], jjA.at[s]
        if s < 16:
            return iiB.at[s - 8], jjB.at[s - 8]
        return iiA.at[s - 16], jjA.at[s - 16]

    pltpu.sync_copy(i2_hbm.at[pl.ds(baser, 8)], iiA)
    pltpu.sync_copy(j2_hbm.at[pl.ds(baser, 8)], jjA)
    ir0, jr0 = idxrow(0)
    ir1, jr1 = idxrow(1)
    issue_in(0, 0, ir0, jr0)
    issue_in(1, 1, ir1, jr1)

    def body(t, carry):
        c0 = t * 16
        r0 = baser + c0
        pltpu.async_copy(i2_hbm.at[pl.ds(r0 + 8, 8)], iiB, semiB)
        pltpu.async_copy(j2_hbm.at[pl.ds(r0 + 8, 8)], jjB, semiB)
        for s in range(16):
            S = s % 2
            cs = c0 + s
            ir, jr = idxrow(s)
            wait_in(S, cs, ir, jr)
            compute(S)
            issue_out(S, ir, jr)
            if s == 6:
                pltpu.make_async_copy(i2_hbm.at[pl.ds(r0 + 8, 8)], iiB,
                                      semiB).wait()
                pltpu.make_async_copy(j2_hbm.at[pl.ds(r0 + 8, 8)], jjB,
                                      semiB).wait()
            if s == 10:
                @pl.when(c0 + 16 < NCH)
                def _():
                    pltpu.async_copy(i2_hbm.at[pl.ds(r0 + 16, 8)], iiA, semiA)
                    pltpu.async_copy(j2_hbm.at[pl.ds(r0 + 16, 8)], jjA, semiA)
            if s == 14:
                @pl.when(c0 + 16 < NCH)
                def _():
                    pltpu.make_async_copy(i2_hbm.at[pl.ds(r0 + 16, 8)], iiA,
                                          semiA).wait()
                    pltpu.make_async_copy(j2_hbm.at[pl.ds(r0 + 16, 8)], jjA,
                                          semiA).wait()
            nir, njr = idxrow(s + 2)

            @pl.when(cs + 2 < NCH)
            def _(S=S, cs=cs, nir=nir, njr=njr):
                wait_out(S, nir, njr)
                issue_in(S, cs + 2, nir, njr)

        return carry

    lax.fori_loop(0, NCH // 16, body, 0)
    wait_out(0, ir0, jr0)
    wait_out(1, ir1, jr1)
    plsc.subcore_barrier()

    def wcopy(q, carry):
        g = ss + 16 * q

        @pl.when(g < NG)
        def _():
            pltpu.sync_copy(agg_sh.at[pl.ds(g * 8, 8)],
                            out_hbm.at[cc, pl.ds(g * 8, 8)])

        return carry

    lax.fori_loop(0, (NG + 15) // 16, wcopy, 0)


def kernel(nxyz, num_atoms, nbr_list, embed, Wef1, bef1, Wef2, bef2, Wnf, bnf,
           Wu1, bu1, Wu2, bu2, Wr1, br1, Wr2, br2):
    del num_atoms
    nbr = nbr_list.astype(jnp.int32)
    spread = (jnp.arange(E_PAD - E, dtype=jnp.int32) * 7) % N
    i_idx = jnp.concatenate([nbr[:, 0], spread])
    j_idx = jnp.concatenate([nbr[:, 1], spread])
    xyzf = jnp.pad(nxyz[:, 1:4], ((0, 0), (0, 1))).reshape(-1)
    z_col = nxyz[:, 0:1]

    def row(b):
        return b.reshape(1, -1)

    dist = _sc_dist2(xyzf, i_idx, j_idx).reshape(E_PAD, 1)
    i2 = i_idx.reshape(E_PAD // KC, KC)
    j2 = j_idx.reshape(E_PAD // KC, KC)

    r, rn = _embed_call(z_col, embed, Wnf[0], row(bnf[0]))

    ws = [_wfilt_call(dist, Wef1[c], row(bef1[c]), Wef2[c], row(bef2[c]))
          for c in range(3)]

    for c in range(3):
        agg = _sc_conv(i2, j2, ws[c], rn)
        if c < 2:
            r, rn = _update_call(agg[0], agg[1], r, Wu1[c], row(bu1[c]),
                                 Wu2[c], row(bu2[c]), Wnf[c + 1],
                                 row(bnf[c + 1]))
        else:
            atomwise = _update_readout_call(agg[0], agg[1], r, Wu1[c],
                                            row(bu1[c]), Wu2[c], row(bu2[c]),
                                            Wr1, row(br1), Wr2,
                                            br2.reshape(1, 1))

    energy = _pool_call(atomwise.reshape(NMOL, NMOL))
    return energy.reshape(NMOL)


# speedup vs baseline: 1.0438x; 1.0438x over previous
import functools

import jax
import jax.numpy as jnp
from jax import lax
from jax.experimental import pallas as pl
from jax.experimental.pallas import tpu as pltpu
from jax.experimental.pallas import tpu_sc as plsc

F32 = jnp.float32

N = 10000
E = 320000
D = 128
G = 50
NMOL = 100
CUTOFF = 5.0
WIDTH = CUTOFF / (G - 1)
LN2 = 0.6931471805599453

NC = 2
NS = 16
NW = NC * NS
EPW = 10240
E_PAD = NW * EPW
K = 128
KD = EPW // K
KC = 40
NCH = EPW // KC
N_PAD = 10112
RPT = N_PAD // NS

BN = 1000
BE_W = 1024


def _ssp(x):
    return jnp.maximum(x, 0.0) + jnp.log(1.0 + jnp.exp(-jnp.abs(x))) - LN2


def _embed_body(z_ref, emb_ref, wn_ref, bn_ref, r_ref, rn_ref):
    z = z_ref[...]
    ids = lax.broadcasted_iota(jnp.int32, (BN, 100), 1).astype(F32)
    oh = (z == ids).astype(F32)
    r = jnp.dot(oh, emb_ref[...], preferred_element_type=F32)
    r_ref[...] = r
    rn_ref[...] = jnp.dot(r, wn_ref[...], preferred_element_type=F32) + bn_ref[...]


def _wfilt_body(d2_ref, w1_ref, b1_ref, w2_ref, b2_ref, out_ref):
    pid = pl.program_id(0)
    d = jnp.sqrt(d2_ref[...])
    offs = lax.broadcasted_iota(jnp.int32, (BE_W, G), 1).astype(F32) * WIDTH
    g = jnp.exp(-0.5 * ((d - offs) / WIDTH) ** 2)
    h = _ssp(jnp.dot(g, w1_ref[...], preferred_element_type=F32) + b1_ref[...])
    w = jnp.dot(h, w2_ref[...], preferred_element_type=F32) + b2_ref[...]
    eid = pid * BE_W + lax.broadcasted_iota(jnp.int32, (BE_W, 1), 0)
    out_ref[...] = jnp.where(eid < E, w, 0.0)


def _update_body(a0_ref, a1_ref, r_ref, wu1_ref, bu1_ref, wu2_ref, bu2_ref,
                 wn_ref, bn_ref, r2_ref, rn_ref):
    agg = a0_ref[...] + a1_ref[...]
    t = _ssp(jnp.dot(agg, wu1_ref[...], preferred_element_type=F32) + bu1_ref[...])
    r2 = r_ref[...] + jnp.dot(t, wu2_ref[...], preferred_element_type=F32) + bu2_ref[...]
    r2_ref[...] = r2
    rn_ref[...] = jnp.dot(r2, wn_ref[...], preferred_element_type=F32) + bn_ref[...]


def _update_readout_body(a0_ref, a1_ref, r_ref, wu1_ref, bu1_ref, wu2_ref,
                         bu2_ref, wr1_ref, br1_ref, wr2_ref, br2_ref, aw_ref):
    agg = a0_ref[...] + a1_ref[...]
    t = _ssp(jnp.dot(agg, wu1_ref[...], preferred_element_type=F32) + bu1_ref[...])
    r2 = r_ref[...] + jnp.dot(t, wu2_ref[...], preferred_element_type=F32) + bu2_ref[...]
    t2 = _ssp(jnp.dot(r2, wr1_ref[...], preferred_element_type=F32) + br1_ref[...])
    aw_ref[...] = jnp.dot(t2, wr2_ref[...], preferred_element_type=F32) + br2_ref[...]


def _pool_body(a_ref, e_ref):
    e_ref[...] = jnp.sum(a_ref[...], axis=1, keepdims=True)


def _full(shape):
    return pl.BlockSpec(shape, lambda i: tuple(0 for _ in shape))


def _embed_call(z_col, emb, wn, bn):
    return pl.pallas_call(
        _embed_body,
        grid=(N // BN,),
        in_specs=[pl.BlockSpec((BN, 1), lambda i: (i, 0)),
                  _full((100, D)), _full((D, D)), _full((1, D))],
        out_specs=[pl.BlockSpec((BN, D), lambda i: (i, 0)),
                   pl.BlockSpec((BN, D), lambda i: (i, 0))],
        out_shape=[jax.ShapeDtypeStruct((N, D), F32),
                   jax.ShapeDtypeStruct((N, D), F32)],
    )(z_col, emb, wn, bn)


def _wfilt_call(dist, w1, b1, w2, b2):
    return pl.pallas_call(
        _wfilt_body,
        grid=(E_PAD // BE_W,),
        in_specs=[pl.BlockSpec((BE_W, 1), lambda i: (i, 0)),
                  _full((G, D)), _full((1, D)), _full((D, D)), _full((1, D))],
        out_specs=pl.BlockSpec((BE_W, D), lambda i: (i, 0)),
        out_shape=jax.ShapeDtypeStruct((E_PAD, D), F32),
    )(dist, w1, b1, w2, b2)


def _update_call(a0, a1, r, wu1, bu1, wu2, bu2, wn, bn):
    return pl.pallas_call(
        _update_body,
        grid=(N // BN,),
        in_specs=[pl.BlockSpec((BN, D), lambda i: (i, 0)),
                  pl.BlockSpec((BN, D), lambda i: (i, 0)),
                  pl.BlockSpec((BN, D), lambda i: (i, 0)),
                  _full((D, D)), _full((1, D)), _full((D, D)), _full((1, D)),
                  _full((D, D)), _full((1, D))],
        out_specs=[pl.BlockSpec((BN, D), lambda i: (i, 0)),
                   pl.BlockSpec((BN, D), lambda i: (i, 0))],
        out_shape=[jax.ShapeDtypeStruct((N, D), F32),
                   jax.ShapeDtypeStruct((N, D), F32)],
    )(a0, a1, r, wu1, bu1, wu2, bu2, wn, bn)


def _update_readout_call(a0, a1, r, wu1, bu1, wu2, bu2, wr1, br1, wr2, br2):
    return pl.pallas_call(
        _update_readout_body,
        grid=(N // BN,),
        in_specs=[pl.BlockSpec((BN, D), lambda i: (i, 0)),
                  pl.BlockSpec((BN, D), lambda i: (i, 0)),
                  pl.BlockSpec((BN, D), lambda i: (i, 0)),
                  _full((D, D)), _full((1, D)), _full((D, D)), _full((1, D)),
                  _full((D, D // 2)), _full((1, D // 2)),
                  _full((D // 2, 1)), _full((1, 1))],
        out_specs=pl.BlockSpec((BN, 1), lambda i: (i, 0)),
        out_shape=jax.ShapeDtypeStruct((N, 1), F32),
    )(a0, a1, r, wu1, bu1, wu2, bu2, wr1, br1, wr2, br2)


def _pool_call(aw):
    return pl.pallas_call(
        _pool_body,
        grid=(1,),
        in_specs=[_full((NMOL, NMOL))],
        out_specs=_full((NMOL, 1)),
        out_shape=jax.ShapeDtypeStruct((NMOL, 1), F32),
    )(aw)


_SC_MESH = plsc.VectorSubcoreMesh(core_axis_name="c", subcore_axis_name="s",
                                  num_cores=NC, num_subcores=NS)


@functools.partial(
    pl.kernel,
    out_type=jax.ShapeDtypeStruct((E_PAD,), F32),
    mesh=_SC_MESH,
    scratch_types=[
        pltpu.VMEM((N * 4,), F32),
        pltpu.VMEM((EPW,), jnp.int32),
        pltpu.VMEM((EPW,), jnp.int32),
        pltpu.VMEM((K,), F32),
    ],
    compiler_params=pltpu.CompilerParams(needs_layout_passes=False),
)
def _sc_dist2(xyz_hbm, i_hbm, j_hbm, d2_hbm, xyz_v, ii_v, jj_v, d_v):
    cc = lax.axis_index("c")
    ss = lax.axis_index("s")
    base = (ss * NC + cc) * EPW
    pltpu.sync_copy(xyz_hbm, xyz_v)
    pltpu.sync_copy(i_hbm.at[pl.ds(base, EPW)], ii_v)
    pltpu.sync_copy(j_hbm.at[pl.ds(base, EPW)], jj_v)

    def chunk(ci, carry):
        off = base + ci * K
        for k in range(8):
            sl = pl.ds(16 * k, 16)
            el = pl.ds(ci * K + 16 * k, 16)
            vi = ii_v[el] * 4
            vj = jj_v[el] * 4
            acc = jnp.zeros((16,), F32)
            for ax in range(3):
                a = plsc.load_gather(xyz_v, [vi + ax])
                b = plsc.load_gather(xyz_v, [vj + ax])
                dd = a - b
                acc = acc + dd * dd
            d_v[sl] = acc
        pltpu.sync_copy(d_v, d2_hbm.at[pl.ds(off, K)])
        return carry

    lax.fori_loop(0, KD, chunk, 0)


@functools.partial(
    pl.kernel,
    out_type=jax.ShapeDtypeStruct((NC, N, D), F32),
    mesh=_SC_MESH,
    scratch_types=[
        pltpu.VMEM((KC, D), F32),
        pltpu.VMEM((KC, D), F32),
        pltpu.VMEM((KC, D), F32),
        pltpu.VMEM((KC, D), F32),
        pltpu.VMEM((KC, D), F32),
        pltpu.VMEM((KC, D), F32),
        pltpu.VMEM((8, KC), jnp.int32),
        pltpu.VMEM((8, KC), jnp.int32),
        pltpu.VMEM((8, KC), jnp.int32),
        pltpu.VMEM((8, KC), jnp.int32),
        pltpu.VMEM_SHARED((N, D), F32),
        pltpu.SemaphoreType.DMA,
        pltpu.SemaphoreType.DMA,
        pltpu.SemaphoreType.DMA,
        pltpu.SemaphoreType.DMA,
        pltpu.SemaphoreType.DMA,
        pltpu.SemaphoreType.DMA,
    ],
)
def _sc_conv(i2_hbm, j2_hbm, w_hbm, rn_hbm, out_hbm,
             rj0, ri0, w0, rj1, ri1, w1, iiA, jjA, iiB, jjB,
             agg_sh, semg0, semg1, sems0, sems1, semiA, semiB):
    sets = ((rj0, ri0, w0, semg0, sems0),
            (rj1, ri1, w1, semg1, sems1))
    cc = lax.axis_index("c")
    ss = lax.axis_index("s")
    base = (ss * NC + cc) * EPW
    baser = (ss * NC + cc) * NCH
    NG = N // 8

    def zrow(t, carry):
        for k in range(8):
            rj0[t, pl.ds(16 * k, 16)] = jnp.zeros((16,), F32)
        return carry

    lax.fori_loop(0, 8, zrow, 0)

    def zcopy(q, carry):
        g = ss + 16 * q

        @pl.when(g < NG)
        def _():
            pltpu.sync_copy(rj0.at[pl.ds(0, 8)], agg_sh.at[pl.ds(g * 8, 8)])

        return carry

    lax.fori_loop(0, (NG + 15) // 16, zcopy, 0)
    plsc.subcore_barrier()

    def issue_in(S, ci, ir, jr):
        rj, ri, w, semg, _ = sets[S]
        off = base + ci * KC
        pltpu.async_copy(rn_hbm.at[jr], rj, semg)
        pltpu.async_copy(rn_hbm.at[ir], ri, semg)
        pltpu.async_copy(w_hbm.at[pl.ds(off, KC)], w, semg)

    def wait_in(S, ci, ir, jr):
        rj, ri, w, semg, _ = sets[S]
        off = base + ci * KC
        pltpu.make_async_copy(rn_hbm.at[jr], rj, semg).wait()
        pltpu.make_async_copy(rn_hbm.at[ir], ri, semg).wait()
        pltpu.make_async_copy(w_hbm.at[pl.ds(off, KC)], w, semg).wait()

    def compute(S):
        rj, ri, w = sets[S][0], sets[S][1], sets[S][2]

        def edges(t, carry):
            for u in range(4):
                e = t * 4 + u
                for k in range(8):
                    sl = pl.ds(16 * k, 16)
                    wv = w[e, sl]
                    rj[e, sl] = rj[e, sl] * wv
                    ri[e, sl] = ri[e, sl] * wv
            return carry

        lax.fori_loop(0, KC // 4, edges, 0)

    def issue_out(S, ir, jr):
        rj, ri, _, _, sems = sets[S]
        pltpu.async_copy(rj, agg_sh.at[ir], sems, add=True)
        pltpu.async_copy(ri, agg_sh.at[jr], sems, add=True)

    def wait_out(S, ir, jr):
        rj, ri, _, _, sems = sets[S]
        pltpu.make_async_copy(rj, agg_sh.at[ir], sems).wait()
        pltpu.make_async_copy(ri, agg_sh.at[jr], sems).wait()

    def idxrow(s):
        if s < 8:
            return iiA.at[s], jjA.at[s]
        if s < 16:
            return iiB.at[s - 8], jjB.at[s - 8]
        return iiA.at[s - 16], jjA.at[s - 16]

    pltpu.sync_copy(i2_hbm.at[pl.ds(baser, 8)], iiA)
    pltpu.sync_copy(j2_hbm.at[pl.ds(baser, 8)], jjA)
    ir0, jr0 = idxrow(0)
    ir1, jr1 = idxrow(1)
    issue_in(0, 0, ir0, jr0)
    issue_in(1, 1, ir1, jr1)

    def body(t, carry):
        c0 = t * 16
        r0 = baser + c0
        pltpu.async_copy(i2_hbm.at[pl.ds(r0 + 8, 8)], iiB, semiB)
        pltpu.async_copy(j2_hbm.at[pl.ds(r0 + 8, 8)], jjB, semiB)
        for s in range(16):
            S = s % 2
            cs = c0 + s
            ir, jr = idxrow(s)
            wait_in(S, cs, ir, jr)
            compute(S)
            issue_out(S, ir, jr)
            if s == 6:
                pltpu.make_async_copy(i2_hbm.at[pl.ds(r0 + 8, 8)], iiB,
                                      semiB).wait()
                pltpu.make_async_copy(j2_hbm.at[pl.ds(r0 + 8, 8)], jjB,
                                      semiB).wait()
            if s == 10:
                @pl.when(c0 + 16 < NCH)
                def _():
                    pltpu.async_copy(i2_hbm.at[pl.ds(r0 + 16, 8)], iiA, semiA)
                    pltpu.async_copy(j2_hbm.at[pl.ds(r0 + 16, 8)], jjA, semiA)
            if s == 14:
                @pl.when(c0 + 16 < NCH)
                def _():
                    pltpu.make_async_copy(i2_hbm.at[pl.ds(r0 + 16, 8)], iiA,
                                          semiA).wait()
                    pltpu.make_async_copy(j2_hbm.at[pl.ds(r0 + 16, 8)], jjA,
                                          semiA).wait()
            nir, njr = idxrow(s + 2)

            @pl.when(cs + 2 < NCH)
            def _(S=S, cs=cs, nir=nir, njr=njr):
                wait_out(S, nir, njr)
                issue_in(S, cs + 2, nir, njr)

        return carry

    lax.fori_loop(0, NCH // 16, body, 0)
    wait_out(0, ir0, jr0)
    wait_out(1, ir1, jr1)
    plsc.subcore_barrier()

    def wcopy(q, carry):
        g = ss + 16 * q

        @pl.when(g < NG)
        def _():
            pltpu.sync_copy(agg_sh.at[pl.ds(g * 8, 8)],
                            out_hbm.at[cc, pl.ds(g * 8, 8)])

        return carry

    lax.fori_loop(0, (NG + 15) // 16, wcopy, 0)


def kernel(nxyz, num_atoms, nbr_list, embed, Wef1, bef1, Wef2, bef2, Wnf, bnf,
           Wu1, bu1, Wu2, bu2, Wr1, br1, Wr2, br2):
    del num_atoms
    nbr = nbr_list.astype(jnp.int32)
    spread = (jnp.arange(E_PAD - E, dtype=jnp.int32) * 7) % N
    i_idx = jnp.concatenate([nbr[:, 0], spread])
    j_idx = jnp.concatenate([nbr[:, 1], spread])
    xyzf = jnp.pad(nxyz[:, 1:4], ((0, 0), (0, 1))).reshape(-1)
    z_col = nxyz[:, 0:1]

    def row(b):
        return b.reshape(1, -1)

    dist = _sc_dist2(xyzf, i_idx, j_idx).reshape(E_PAD, 1)
    i2 = i_idx.reshape(E_PAD // KC, KC)
    j2 = j_idx.reshape(E_PAD // KC, KC)

    r, rn = _embed_call(z_col, embed, Wnf[0], row(bnf[0]))

    ws = [_wfilt_call(dist, Wef1[c], row(bef1[c]), Wef2[c], row(bef2[c]))
          for c in range(3)]

    for c in range(3):
        agg = _sc_conv(i2, j2, ws[c], rn)
        if c < 2:
            r, rn = _update_call(agg[0], agg[1], r, Wu1[c], row(bu1[c]),
                                 Wu2[c], row(bu2[c]), Wnf[c + 1],
                                 row(bnf[c + 1]))
        else:
            atomwise = _update_readout_call(agg[0], agg[1], r, Wu1[c],
                                            row(bu1[c]), Wu2[c], row(bu2[c]),
                                            Wr1, row(br1), Wr2,
                                            br2.reshape(1, 1))

    energy = _pool_call(atomwise.reshape(NMOL, NMOL))
    return energy.reshape(NMOL)
